# baseline (speedup 1.0000x reference)
import jax
import jax.numpy as jnp
from jax import lax
from jax.experimental import pallas as pl
from jax.experimental.pallas import tpu as pltpu
from jax.experimental.pallas import tpu_sc as plsc

_NC = 2
_NS = 16
_L = 16


def _make_iwt_sc(B, H, W, C4):
    n = C4 // 4
    R = B * H
    NW = _NC * _NS
    assert R % NW == 0 and W % 2 == 0 and n % _L == 0
    rows_per_worker = R // NW
    Wh = W // 2

    mesh = plsc.VectorSubcoreMesh(
        core_axis_name="c", subcore_axis_name="s",
        num_cores=_NC, num_subcores=_NS)

    def butterfly(xin, yev, yod):
        @plsc.parallel_loop(0, Wh, unroll=4)
        def _(w):
            for k in range(n // _L):
                x1 = xin[w, pl.ds(k * _L, _L)]
                x2 = xin[w, pl.ds(n + k * _L, _L)]
                x3 = xin[w, pl.ds(2 * n + k * _L, _L)]
                x4 = xin[w, pl.ds(3 * n + k * _L, _L)]
                s12 = x1 + x2
                d12 = x1 - x2
                s34 = x3 + x4
                d34 = x3 - x4
                yev[2 * w, pl.ds(k * _L, _L)] = (d12 - d34) * 0.5
                yev[2 * w + 1, pl.ds(k * _L, _L)] = (s12 - s34) * 0.5
                yod[2 * w, pl.ds(k * _L, _L)] = (d12 + d34) * 0.5
                yod[2 * w + 1, pl.ds(k * _L, _L)] = (s12 + s34) * 0.5

    def body(x_hbm, y_hbm,
             xa, xb, eva, evb, oda, odb,
             sia, sib, soa, sob):
        wid = lax.axis_index("s") * _NC + lax.axis_index("c")
        row0 = wid * rows_per_worker

        def split(i):
            return i // H, i % H

        def in_cp(i, half, buf, sem):
            b, h = split(i)
            return pltpu.make_async_copy(
                x_hbm.at[b, h, pl.ds(half * Wh, Wh)], buf, sem)

        def out_cp(i, parity, half, buf, sem):
            b, h = split(i)
            return pltpu.make_async_copy(
                buf, y_hbm.at[b, 2 * h + parity, pl.ds(half * W, W)], sem)

        in_cp(row0, 0, xa, sia).start()

        def row_loop(k, carry):
            i = row0 + k
            in_cp(i, 0, xa, sia).wait()
            in_cp(i, 1, xb, sib).start()

            @pl.when(k > 0)
            def _():
                out_cp(i - 1, 0, 0, eva, soa).wait()
                out_cp(i - 1, 1, 0, oda, soa).wait()

            butterfly(xa, eva, oda)
            out_cp(i, 0, 0, eva, soa).start()
            out_cp(i, 1, 0, oda, soa).start()

            in_cp(i, 1, xb, sib).wait()

            @pl.when(k < rows_per_worker - 1)
            def _():
                in_cp(i + 1, 0, xa, sia).start()

            @pl.when(k > 0)
            def _():
                out_cp(i - 1, 0, 1, evb, sob).wait()
                out_cp(i - 1, 1, 1, odb, sob).wait()

            butterfly(xb, evb, odb)
            out_cp(i, 0, 1, evb, sob).start()
            out_cp(i, 1, 1, odb, sob).start()
            return carry

        lax.fori_loop(0, rows_per_worker, row_loop, 0)
        last = row0 + rows_per_worker - 1
        out_cp(last, 0, 0, eva, soa).wait()
        out_cp(last, 1, 0, oda, soa).wait()
        out_cp(last, 0, 1, evb, sob).wait()
        out_cp(last, 1, 1, odb, sob).wait()

    return pl.kernel(
        body,
        out_type=jax.ShapeDtypeStruct((B, 2 * H, 2 * W, n), jnp.float32),
        mesh=mesh,
        scratch_types=[
            pltpu.VMEM((Wh, C4), jnp.float32),
            pltpu.VMEM((Wh, C4), jnp.float32),
            pltpu.VMEM((W, n), jnp.float32),
            pltpu.VMEM((W, n), jnp.float32),
            pltpu.VMEM((W, n), jnp.float32),
            pltpu.VMEM((W, n), jnp.float32),
            pltpu.SemaphoreType.DMA,
            pltpu.SemaphoreType.DMA,
            pltpu.SemaphoreType.DMA,
            pltpu.SemaphoreType.DMA,
        ],
    )


def _make_relayout_tc(B, H2, W2, n, Hb=8):
    def tbody(x_ref, o_ref):
        eye = jnp.eye(W2, dtype=jnp.float32)
        o_ref[0] = lax.dot_general(
            x_ref[0], eye, (((1,), (0,)), ((), ())),
            preferred_element_type=jnp.float32)

    return pl.pallas_call(
        tbody,
        grid=(B, H2 // Hb),
        in_specs=[pl.BlockSpec((1, Hb, W2, n), lambda b, h: (b, h, 0, 0))],
        out_specs=pl.BlockSpec((1, Hb, n, W2), lambda b, h: (b, h, 0, 0)),
        out_shape=jax.ShapeDtypeStruct((B, H2, n, W2), jnp.float32),
    )


def kernel(inputs):
    B, H, W, C4 = inputs.shape
    n = C4 // 4
    y = _make_iwt_sc(B, H, W, C4)(inputs)
    z = _make_relayout_tc(B, 2 * H, 2 * W, n)(y)
    return jnp.transpose(z, (0, 1, 3, 2))


# speedup vs baseline: 1.1294x; 1.1294x over previous
import jax
import jax.numpy as jnp
from jax import lax
from jax.experimental import pallas as pl
from jax.experimental.pallas import tpu as pltpu
from jax.experimental.pallas import tpu_sc as plsc

_NC = 2
_NS = 16
_L = 16


def _make_iwt_sc(B, H, W, C4, b0, Bs):
    n = C4 // 4
    R = Bs * H
    NW = _NC * _NS
    assert R % NW == 0 and W % 2 == 0 and n % _L == 0
    rows_per_worker = R // NW
    Wh = W // 2

    mesh = plsc.VectorSubcoreMesh(
        core_axis_name="c", subcore_axis_name="s",
        num_cores=_NC, num_subcores=_NS)

    def butterfly(xin, yev, yod):
        @plsc.parallel_loop(0, Wh, unroll=4)
        def _(w):
            for k in range(n // _L):
                x1 = xin[w, pl.ds(k * _L, _L)]
                x2 = xin[w, pl.ds(n + k * _L, _L)]
                x3 = xin[w, pl.ds(2 * n + k * _L, _L)]
                x4 = xin[w, pl.ds(3 * n + k * _L, _L)]
                s12 = x1 + x2
                d12 = x1 - x2
                s34 = x3 + x4
                d34 = x3 - x4
                yev[2 * w, pl.ds(k * _L, _L)] = (d12 - d34) * 0.5
                yev[2 * w + 1, pl.ds(k * _L, _L)] = (s12 - s34) * 0.5
                yod[2 * w, pl.ds(k * _L, _L)] = (d12 + d34) * 0.5
                yod[2 * w + 1, pl.ds(k * _L, _L)] = (s12 + s34) * 0.5

    def body(x_hbm, y_hbm,
             xa, xb, eva, evb, oda, odb,
             sia, sib, soa, sob):
        wid = lax.axis_index("s") * _NC + lax.axis_index("c")
        row0 = wid * rows_per_worker

        def split(i):
            return b0 + i // H, i % H

        def in_cp(i, half, buf, sem):
            b, h = split(i)
            return pltpu.make_async_copy(
                x_hbm.at[b, h, pl.ds(half * Wh, Wh)], buf, sem)

        def out_cp(i, parity, half, buf, sem):
            b, h = split(i)
            return pltpu.make_async_copy(
                buf, y_hbm.at[b - b0, 2 * h + parity, pl.ds(half * W, W)], sem)

        in_cp(row0, 0, xa, sia).start()

        def row_loop(k, carry):
            i = row0 + k
            in_cp(i, 0, xa, sia).wait()
            in_cp(i, 1, xb, sib).start()

            @pl.when(k > 0)
            def _():
                out_cp(i - 1, 0, 0, eva, soa).wait()
                out_cp(i - 1, 1, 0, oda, soa).wait()

            butterfly(xa, eva, oda)
            out_cp(i, 0, 0, eva, soa).start()
            out_cp(i, 1, 0, oda, soa).start()

            in_cp(i, 1, xb, sib).wait()

            @pl.when(k < rows_per_worker - 1)
            def _():
                in_cp(i + 1, 0, xa, sia).start()

            @pl.when(k > 0)
            def _():
                out_cp(i - 1, 0, 1, evb, sob).wait()
                out_cp(i - 1, 1, 1, odb, sob).wait()

            butterfly(xb, evb, odb)
            out_cp(i, 0, 1, evb, sob).start()
            out_cp(i, 1, 1, odb, sob).start()
            return carry

        lax.fori_loop(0, rows_per_worker, row_loop, 0)
        last = row0 + rows_per_worker - 1
        out_cp(last, 0, 0, eva, soa).wait()
        out_cp(last, 1, 0, oda, soa).wait()
        out_cp(last, 0, 1, evb, sob).wait()
        out_cp(last, 1, 1, odb, sob).wait()

    return pl.kernel(
        body,
        out_type=jax.ShapeDtypeStruct((Bs, 2 * H, 2 * W, n), jnp.float32),
        mesh=mesh,
        scratch_types=[
            pltpu.VMEM((Wh, C4), jnp.float32),
            pltpu.VMEM((Wh, C4), jnp.float32),
            pltpu.VMEM((W, n), jnp.float32),
            pltpu.VMEM((W, n), jnp.float32),
            pltpu.VMEM((W, n), jnp.float32),
            pltpu.VMEM((W, n), jnp.float32),
            pltpu.SemaphoreType.DMA,
            pltpu.SemaphoreType.DMA,
            pltpu.SemaphoreType.DMA,
            pltpu.SemaphoreType.DMA,
        ],
    )


def _make_relayout_tc(B, H2, W2, n, g, Bs, first, Hb=8):
    out_shape = jax.ShapeDtypeStruct((B, H2, n, W2), jnp.float32)
    out_spec = pl.BlockSpec((1, Hb, n, W2), lambda b, h: (g * Bs + b, h, 0, 0))
    in_spec = pl.BlockSpec((1, Hb, W2, n), lambda b, h: (b, h, 0, 0))

    if first:
        def tbody0(x_ref, o_ref):
            o_ref[0] = jnp.transpose(x_ref[0], (0, 2, 1))

        return pl.pallas_call(
            tbody0,
            grid=(Bs, H2 // Hb),
            in_specs=[in_spec],
            out_specs=out_spec,
            out_shape=out_shape,
        )

    def tbody(acc_ref, x_ref, o_ref):
        del acc_ref
        o_ref[0] = jnp.transpose(x_ref[0], (0, 2, 1))

    return pl.pallas_call(
        tbody,
        grid=(Bs, H2 // Hb),
        in_specs=[pl.BlockSpec(memory_space=pl.ANY), in_spec],
        out_specs=out_spec,
        out_shape=out_shape,
        input_output_aliases={0: 0},
    )


def kernel(inputs):
    B, H, W, C4 = inputs.shape
    n = C4 // 4
    NW = _NC * _NS
    Bs = next(s for s in range(1, B + 1)
              if B % s == 0 and (s * H) % NW == 0)
    acc = None
    for g in range(B // Bs):
        y = _make_iwt_sc(B, H, W, C4, g * Bs, Bs)(inputs)
        if acc is None:
            acc = _make_relayout_tc(B, 2 * H, 2 * W, n, g, Bs, True)(y)
        else:
            acc = _make_relayout_tc(B, 2 * H, 2 * W, n, g, Bs, False)(acc, y)
    return jnp.transpose(acc, (0, 1, 3, 2))


# speedup vs baseline: 1.3432x; 1.1894x over previous
import jax
import jax.numpy as jnp
from jax import lax
from jax.experimental import pallas as pl
from jax.experimental.pallas import tpu as pltpu
from jax.experimental.pallas import tpu_sc as plsc

_NC = 2
_NS = 16
_L = 16


def _make_iwt_sc(B, H, W, C4, b0, Bs):
    n = C4 // 4
    R = Bs * H
    NW = _NC * _NS
    assert R % NW == 0 and W % 2 == 0 and n % _L == 0
    rows_per_worker = R // NW
    Wh = W // 2

    mesh = plsc.VectorSubcoreMesh(
        core_axis_name="c", subcore_axis_name="s",
        num_cores=_NC, num_subcores=_NS)

    def butterfly(xin, yev, yod):
        @plsc.parallel_loop(0, Wh, unroll=4)
        def _(w):
            for k in range(n // _L):
                x1 = xin[w, pl.ds(k * _L, _L)]
                x2 = xin[w, pl.ds(n + k * _L, _L)]
                x3 = xin[w, pl.ds(2 * n + k * _L, _L)]
                x4 = xin[w, pl.ds(3 * n + k * _L, _L)]
                s12 = x1 + x2
                d12 = x1 - x2
                s34 = x3 + x4
                d34 = x3 - x4
                yev[2 * w, pl.ds(k * _L, _L)] = (d12 - d34) * 0.5
                yev[2 * w + 1, pl.ds(k * _L, _L)] = (s12 - s34) * 0.5
                yod[2 * w, pl.ds(k * _L, _L)] = (d12 + d34) * 0.5
                yod[2 * w + 1, pl.ds(k * _L, _L)] = (s12 + s34) * 0.5

    def body(x_hbm, y_hbm,
             xa, xb, eva, evb, oda, odb,
             sia, sib, soa, sob):
        wid = lax.axis_index("s") * _NC + lax.axis_index("c")
        row0 = wid * rows_per_worker

        def split(i):
            return b0 + i // H, i % H

        def in_cp(i, half, buf, sem):
            b, h = split(i)
            return pltpu.make_async_copy(
                x_hbm.at[b, h, pl.ds(half * Wh, Wh)], buf, sem)

        def out_cp(i, parity, half, buf, sem):
            b, h = split(i)
            return pltpu.make_async_copy(
                buf, y_hbm.at[b - b0, 2 * h + parity, pl.ds(half * W, W)], sem)

        in_cp(row0, 0, xa, sia).start()

        def row_loop(k, carry):
            i = row0 + k
            in_cp(i, 0, xa, sia).wait()
            in_cp(i, 1, xb, sib).start()

            @pl.when(k > 0)
            def _():
                out_cp(i - 1, 0, 0, eva, soa).wait()
                out_cp(i - 1, 1, 0, oda, soa).wait()

            butterfly(xa, eva, oda)
            out_cp(i, 0, 0, eva, soa).start()
            out_cp(i, 1, 0, oda, soa).start()

            in_cp(i, 1, xb, sib).wait()

            @pl.when(k < rows_per_worker - 1)
            def _():
                in_cp(i + 1, 0, xa, sia).start()

            @pl.when(k > 0)
            def _():
                out_cp(i - 1, 0, 1, evb, sob).wait()
                out_cp(i - 1, 1, 1, odb, sob).wait()

            butterfly(xb, evb, odb)
            out_cp(i, 0, 1, evb, sob).start()
            out_cp(i, 1, 1, odb, sob).start()
            return carry

        lax.fori_loop(0, rows_per_worker, row_loop, 0)
        last = row0 + rows_per_worker - 1
        out_cp(last, 0, 0, eva, soa).wait()
        out_cp(last, 1, 0, oda, soa).wait()
        out_cp(last, 0, 1, evb, sob).wait()
        out_cp(last, 1, 1, odb, sob).wait()

    return pl.kernel(
        body,
        out_type=jax.ShapeDtypeStruct((Bs, 2 * H, 2 * W, n), jnp.float32),
        mesh=mesh,
        scratch_types=[
            pltpu.VMEM((Wh, C4), jnp.float32),
            pltpu.VMEM((Wh, C4), jnp.float32),
            pltpu.VMEM((W, n), jnp.float32),
            pltpu.VMEM((W, n), jnp.float32),
            pltpu.VMEM((W, n), jnp.float32),
            pltpu.VMEM((W, n), jnp.float32),
            pltpu.SemaphoreType.DMA,
            pltpu.SemaphoreType.DMA,
            pltpu.SemaphoreType.DMA,
            pltpu.SemaphoreType.DMA,
        ],
    )


def _make_relayout_tc(B, H2, W2, n, g, Bs, first, Hb=16):
    out_shape = jax.ShapeDtypeStruct((B, H2, n, W2), jnp.float32)
    out_spec = pl.BlockSpec((1, Hb, n, W2), lambda b, h: (g * Bs + b, h, 0, 0))
    in_spec = pl.BlockSpec((1, Hb, W2, n), lambda b, h: (b, h, 0, 0))

    if first:
        def tbody0(x_ref, o_ref):
            o_ref[0] = jnp.transpose(x_ref[0], (0, 2, 1))

        return pl.pallas_call(
            tbody0,
            grid=(Bs, H2 // Hb),
            in_specs=[in_spec],
            out_specs=out_spec,
            out_shape=out_shape,
        )

    def tbody(acc_ref, x_ref, o_ref):
        del acc_ref
        o_ref[0] = jnp.transpose(x_ref[0], (0, 2, 1))

    return pl.pallas_call(
        tbody,
        grid=(Bs, H2 // Hb),
        in_specs=[pl.BlockSpec(memory_space=pl.ANY), in_spec],
        out_specs=out_spec,
        out_shape=out_shape,
        input_output_aliases={0: 0},
    )


def kernel(inputs):
    B, H, W, C4 = inputs.shape
    n = C4 // 4
    NW = _NC * _NS
    Bs = next(s for s in range(1, B + 1)
              if B % s == 0 and (s * H) % NW == 0)
    acc = None
    for g in range(B // Bs):
        y = _make_iwt_sc(B, H, W, C4, g * Bs, Bs)(inputs)
        if acc is None:
            acc = _make_relayout_tc(B, 2 * H, 2 * W, n, g, Bs, True)(y)
        else:
            acc = _make_relayout_tc(B, 2 * H, 2 * W, n, g, Bs, False)(acc, y)
    return jnp.transpose(acc, (0, 1, 3, 2))


# speedup vs baseline: 1.3974x; 1.0403x over previous
import jax
import jax.numpy as jnp
from jax import lax
from jax.experimental import pallas as pl
from jax.experimental.pallas import tpu as pltpu
from jax.experimental.pallas import tpu_sc as plsc

_NC = 2
_NS = 16
_L = 16


def _make_iwt_sc(B, H, W, C4, b0, Bs):
    n = C4 // 4
    R = Bs * H
    NW = _NC * _NS
    assert R % NW == 0 and W % 2 == 0 and n % _L == 0
    rows_per_worker = R // NW
    Wh = W // 2

    mesh = plsc.VectorSubcoreMesh(
        core_axis_name="c", subcore_axis_name="s",
        num_cores=_NC, num_subcores=_NS)

    def butterfly(xin, yev, yod):
        @plsc.parallel_loop(0, Wh, unroll=4)
        def _(w):
            for k in range(n // _L):
                x1 = xin[w, pl.ds(k * _L, _L)]
                x2 = xin[w, pl.ds(n + k * _L, _L)]
                x3 = xin[w, pl.ds(2 * n + k * _L, _L)]
                x4 = xin[w, pl.ds(3 * n + k * _L, _L)]
                s12 = x1 + x2
                d12 = x1 - x2
                s34 = x3 + x4
                d34 = x3 - x4
                yev[2 * w, pl.ds(k * _L, _L)] = (d12 - d34) * 0.5
                yev[2 * w + 1, pl.ds(k * _L, _L)] = (s12 - s34) * 0.5
                yod[2 * w, pl.ds(k * _L, _L)] = (d12 + d34) * 0.5
                yod[2 * w + 1, pl.ds(k * _L, _L)] = (s12 + s34) * 0.5

    def body(x_hbm, y_hbm,
             xa, xb, eva, evb, oda, odb,
             sia, sib, soa, sob):
        wid = lax.axis_index("s") * _NC + lax.axis_index("c")
        row0 = wid * rows_per_worker

        def split(i):
            return b0 + i // H, i % H

        def in_cp(i, half, buf, sem):
            b, h = split(i)
            return pltpu.make_async_copy(
                x_hbm.at[b, h, pl.ds(half * Wh, Wh)], buf, sem)

        def out_cp(i, parity, half, buf, sem):
            b, h = split(i)
            return pltpu.make_async_copy(
                buf, y_hbm.at[b - b0, 2 * h + parity, pl.ds(half * W, W)], sem)

        in_cp(row0, 0, xa, sia).start()

        def row_loop(k, carry):
            i = row0 + k
            in_cp(i, 0, xa, sia).wait()
            in_cp(i, 1, xb, sib).start()

            @pl.when(k > 0)
            def _():
                out_cp(i - 1, 0, 0, eva, soa).wait()
                out_cp(i - 1, 1, 0, oda, soa).wait()

            butterfly(xa, eva, oda)
            out_cp(i, 0, 0, eva, soa).start()
            out_cp(i, 1, 0, oda, soa).start()

            in_cp(i, 1, xb, sib).wait()

            @pl.when(k < rows_per_worker - 1)
            def _():
                in_cp(i + 1, 0, xa, sia).start()

            @pl.when(k > 0)
            def _():
                out_cp(i - 1, 0, 1, evb, sob).wait()
                out_cp(i - 1, 1, 1, odb, sob).wait()

            butterfly(xb, evb, odb)
            out_cp(i, 0, 1, evb, sob).start()
            out_cp(i, 1, 1, odb, sob).start()
            return carry

        lax.fori_loop(0, rows_per_worker, row_loop, 0)
        last = row0 + rows_per_worker - 1
        out_cp(last, 0, 0, eva, soa).wait()
        out_cp(last, 1, 0, oda, soa).wait()
        out_cp(last, 0, 1, evb, sob).wait()
        out_cp(last, 1, 1, odb, sob).wait()

    return pl.kernel(
        body,
        out_type=jax.ShapeDtypeStruct((Bs, 2 * H, 2 * W, n), jnp.float32),
        mesh=mesh,
        scratch_types=[
            pltpu.VMEM((Wh, C4), jnp.float32),
            pltpu.VMEM((Wh, C4), jnp.float32),
            pltpu.VMEM((W, n), jnp.float32),
            pltpu.VMEM((W, n), jnp.float32),
            pltpu.VMEM((W, n), jnp.float32),
            pltpu.VMEM((W, n), jnp.float32),
            pltpu.SemaphoreType.DMA,
            pltpu.SemaphoreType.DMA,
            pltpu.SemaphoreType.DMA,
            pltpu.SemaphoreType.DMA,
        ],
    )


def _make_relayout_tc(B, H2, W2, n, g, Bs, first, Hb=32):
    out_shape = jax.ShapeDtypeStruct((B, H2, n, W2), jnp.float32)
    out_spec = pl.BlockSpec((1, Hb, n, W2), lambda b, h: (g * Bs + b, h, 0, 0))
    in_spec = pl.BlockSpec((1, Hb, W2, n), lambda b, h: (b, h, 0, 0))

    if first:
        def tbody0(x_ref, o_ref):
            o_ref[0] = jnp.transpose(x_ref[0], (0, 2, 1))

        return pl.pallas_call(
            tbody0,
            grid=(Bs, H2 // Hb),
            in_specs=[in_spec],
            out_specs=out_spec,
            out_shape=out_shape,
        )

    def tbody(acc_ref, x_ref, o_ref):
        del acc_ref
        o_ref[0] = jnp.transpose(x_ref[0], (0, 2, 1))

    return pl.pallas_call(
        tbody,
        grid=(Bs, H2 // Hb),
        in_specs=[pl.BlockSpec(memory_space=pl.ANY), in_spec],
        out_specs=out_spec,
        out_shape=out_shape,
        input_output_aliases={0: 0},
    )


def kernel(inputs):
    B, H, W, C4 = inputs.shape
    n = C4 // 4
    NW = _NC * _NS
    Bs = next(s for s in range(1, B + 1)
              if B % s == 0 and (s * H) % NW == 0)
    acc = None
    for g in range(B // Bs):
        y = _make_iwt_sc(B, H, W, C4, g * Bs, Bs)(inputs)
        if acc is None:
            acc = _make_relayout_tc(B, 2 * H, 2 * W, n, g, Bs, True)(y)
        else:
            acc = _make_relayout_tc(B, 2 * H, 2 * W, n, g, Bs, False)(acc, y)
    return jnp.transpose(acc, (0, 1, 3, 2))


# speedup vs baseline: 1.4082x; 1.0077x over previous
import jax
import jax.numpy as jnp
from jax import lax
from jax.experimental import pallas as pl
from jax.experimental.pallas import tpu as pltpu
from jax.experimental.pallas import tpu_sc as plsc

_NC = 2
_NS = 16
_L = 16


def _make_iwt_sc(B, H, W, C4, b0, Bs):
    n = C4 // 4
    R = Bs * H
    NW = _NC * _NS
    assert R % NW == 0 and W % 2 == 0 and n % _L == 0
    rows_per_worker = R // NW
    Wh = W // 2

    mesh = plsc.VectorSubcoreMesh(
        core_axis_name="c", subcore_axis_name="s",
        num_cores=_NC, num_subcores=_NS)

    def butterfly(xin, yev, yod):
        @plsc.parallel_loop(0, Wh, unroll=4)
        def _(w):
            for k in range(n // _L):
                x1 = xin[w, pl.ds(k * _L, _L)]
                x2 = xin[w, pl.ds(n + k * _L, _L)]
                x3 = xin[w, pl.ds(2 * n + k * _L, _L)]
                x4 = xin[w, pl.ds(3 * n + k * _L, _L)]
                s12 = x1 + x2
                d12 = x1 - x2
                s34 = x3 + x4
                d34 = x3 - x4
                yev[2 * w, pl.ds(k * _L, _L)] = (d12 - d34) * 0.5
                yev[2 * w + 1, pl.ds(k * _L, _L)] = (s12 - s34) * 0.5
                yod[2 * w, pl.ds(k * _L, _L)] = (d12 + d34) * 0.5
                yod[2 * w + 1, pl.ds(k * _L, _L)] = (s12 + s34) * 0.5

    def body(x_hbm, y_hbm,
             xa, xb, eva, evb, oda, odb,
             sia, sib, soa, sob):
        wid = lax.axis_index("s") * _NC + lax.axis_index("c")
        row0 = wid * rows_per_worker

        def split(i):
            return b0 + i // H, i % H

        def in_cp(i, half, buf, sem):
            b, h = split(i)
            return pltpu.make_async_copy(
                x_hbm.at[b, h, pl.ds(half * Wh, Wh)], buf, sem)

        def out_cp(i, parity, half, buf, sem):
            b, h = split(i)
            return pltpu.make_async_copy(
                buf, y_hbm.at[b - b0, 2 * h + parity, pl.ds(half * W, W)], sem)

        in_cp(row0, 0, xa, sia).start()

        def row_loop(k, carry):
            i = row0 + k
            in_cp(i, 0, xa, sia).wait()
            in_cp(i, 1, xb, sib).start()

            @pl.when(k > 0)
            def _():
                out_cp(i - 1, 0, 0, eva, soa).wait()
                out_cp(i - 1, 1, 0, oda, soa).wait()

            butterfly(xa, eva, oda)
            out_cp(i, 0, 0, eva, soa).start()
            out_cp(i, 1, 0, oda, soa).start()

            in_cp(i, 1, xb, sib).wait()

            @pl.when(k < rows_per_worker - 1)
            def _():
                in_cp(i + 1, 0, xa, sia).start()

            @pl.when(k > 0)
            def _():
                out_cp(i - 1, 0, 1, evb, sob).wait()
                out_cp(i - 1, 1, 1, odb, sob).wait()

            butterfly(xb, evb, odb)
            out_cp(i, 0, 1, evb, sob).start()
            out_cp(i, 1, 1, odb, sob).start()
            return carry

        lax.fori_loop(0, rows_per_worker, row_loop, 0)
        last = row0 + rows_per_worker - 1
        out_cp(last, 0, 0, eva, soa).wait()
        out_cp(last, 1, 0, oda, soa).wait()
        out_cp(last, 0, 1, evb, sob).wait()
        out_cp(last, 1, 1, odb, sob).wait()

    return pl.kernel(
        body,
        out_type=jax.ShapeDtypeStruct((Bs, 2 * H, 2 * W, n), jnp.float32),
        mesh=mesh,
        scratch_types=[
            pltpu.VMEM((Wh, C4), jnp.float32),
            pltpu.VMEM((Wh, C4), jnp.float32),
            pltpu.VMEM((W, n), jnp.float32),
            pltpu.VMEM((W, n), jnp.float32),
            pltpu.VMEM((W, n), jnp.float32),
            pltpu.VMEM((W, n), jnp.float32),
            pltpu.SemaphoreType.DMA,
            pltpu.SemaphoreType.DMA,
            pltpu.SemaphoreType.DMA,
            pltpu.SemaphoreType.DMA,
        ],
    )


def _make_relayout_tc(B, H2, W2, n, g, Bs, first, Hb=56):
    out_shape = jax.ShapeDtypeStruct((B, H2, n, W2), jnp.float32)
    out_spec = pl.BlockSpec((1, Hb, n, W2), lambda b, h: (g * Bs + b, h, 0, 0))
    in_spec = pl.BlockSpec((1, Hb, W2, n), lambda b, h: (b, h, 0, 0))

    if first:
        def tbody0(x_ref, o_ref):
            o_ref[0] = jnp.transpose(x_ref[0], (0, 2, 1))

        return pl.pallas_call(
            tbody0,
            grid=(Bs, H2 // Hb),
            in_specs=[in_spec],
            out_specs=out_spec,
            out_shape=out_shape,
        )

    def tbody(acc_ref, x_ref, o_ref):
        del acc_ref
        o_ref[0] = jnp.transpose(x_ref[0], (0, 2, 1))

    return pl.pallas_call(
        tbody,
        grid=(Bs, H2 // Hb),
        in_specs=[pl.BlockSpec(memory_space=pl.ANY), in_spec],
        out_specs=out_spec,
        out_shape=out_shape,
        input_output_aliases={0: 0},
    )


def kernel(inputs):
    B, H, W, C4 = inputs.shape
    n = C4 // 4
    NW = _NC * _NS
    Bs = next(s for s in range(1, B + 1)
              if B % s == 0 and (s * H) % NW == 0)
    acc = None
    for g in range(B // Bs):
        y = _make_iwt_sc(B, H, W, C4, g * Bs, Bs)(inputs)
        if acc is None:
            acc = _make_relayout_tc(B, 2 * H, 2 * W, n, g, Bs, True)(y)
        else:
            acc = _make_relayout_tc(B, 2 * H, 2 * W, n, g, Bs, False)(acc, y)
    return jnp.transpose(acc, (0, 1, 3, 2))


# speedup vs baseline: 1.4225x; 1.0102x over previous
import jax
import jax.numpy as jnp
from jax import lax
from jax.experimental import pallas as pl
from jax.experimental.pallas import tpu as pltpu
from jax.experimental.pallas import tpu_sc as plsc

_NC = 2
_NS = 16
_L = 16


def _make_iwt_sc(B, H, W, C4, b0, Bs):
    n = C4 // 4
    R = Bs * H
    NW = _NC * _NS
    assert R % NW == 0 and W % 2 == 0 and n % _L == 0
    rows_per_worker = R // NW
    Wh = W // 2

    mesh = plsc.VectorSubcoreMesh(
        core_axis_name="c", subcore_axis_name="s",
        num_cores=_NC, num_subcores=_NS)

    def butterfly(xin, yev, yod):
        @plsc.parallel_loop(0, Wh, unroll=4)
        def _(w):
            for k in range(n // _L):
                x1 = xin[w, pl.ds(k * _L, _L)]
                x2 = xin[w, pl.ds(n + k * _L, _L)]
                x3 = xin[w, pl.ds(2 * n + k * _L, _L)]
                x4 = xin[w, pl.ds(3 * n + k * _L, _L)]
                s12 = x1 + x2
                d12 = x1 - x2
                s34 = x3 + x4
                d34 = x3 - x4
                yev[2 * w, pl.ds(k * _L, _L)] = (d12 - d34) * 0.5
                yev[2 * w + 1, pl.ds(k * _L, _L)] = (s12 - s34) * 0.5
                yod[2 * w, pl.ds(k * _L, _L)] = (d12 + d34) * 0.5
                yod[2 * w + 1, pl.ds(k * _L, _L)] = (s12 + s34) * 0.5

    def body(x_hbm, y_hbm,
             xa, xb, eva, evb, oda, odb,
             sia, sib, soa, sob):
        wid = lax.axis_index("s") * _NC + lax.axis_index("c")
        row0 = wid * rows_per_worker

        def split(i):
            return b0 + i // H, i % H

        def in_cp(i, half, buf, sem):
            b, h = split(i)
            return pltpu.make_async_copy(
                x_hbm.at[b, h, pl.ds(half * Wh, Wh)], buf, sem)

        def out_cp(i, parity, half, buf, sem):
            b, h = split(i)
            return pltpu.make_async_copy(
                buf, y_hbm.at[b - b0, 2 * h + parity, pl.ds(half * W, W)], sem)

        in_cp(row0, 0, xa, sia).start()

        def row_loop(k, carry):
            i = row0 + k
            in_cp(i, 0, xa, sia).wait()
            in_cp(i, 1, xb, sib).start()

            @pl.when(k > 0)
            def _():
                out_cp(i - 1, 0, 0, eva, soa).wait()
                out_cp(i - 1, 1, 0, oda, soa).wait()

            butterfly(xa, eva, oda)
            out_cp(i, 0, 0, eva, soa).start()
            out_cp(i, 1, 0, oda, soa).start()

            in_cp(i, 1, xb, sib).wait()

            @pl.when(k < rows_per_worker - 1)
            def _():
                in_cp(i + 1, 0, xa, sia).start()

            @pl.when(k > 0)
            def _():
                out_cp(i - 1, 0, 1, evb, sob).wait()
                out_cp(i - 1, 1, 1, odb, sob).wait()

            butterfly(xb, evb, odb)
            out_cp(i, 0, 1, evb, sob).start()
            out_cp(i, 1, 1, odb, sob).start()
            return carry

        lax.fori_loop(0, rows_per_worker, row_loop, 0)
        last = row0 + rows_per_worker - 1
        out_cp(last, 0, 0, eva, soa).wait()
        out_cp(last, 1, 0, oda, soa).wait()
        out_cp(last, 0, 1, evb, sob).wait()
        out_cp(last, 1, 1, odb, sob).wait()

    return pl.kernel(
        body,
        out_type=jax.ShapeDtypeStruct((Bs, 2 * H, 2 * W, n), jnp.float32),
        mesh=mesh,
        scratch_types=[
            pltpu.VMEM((Wh, C4), jnp.float32),
            pltpu.VMEM((Wh, C4), jnp.float32),
            pltpu.VMEM((W, n), jnp.float32),
            pltpu.VMEM((W, n), jnp.float32),
            pltpu.VMEM((W, n), jnp.float32),
            pltpu.VMEM((W, n), jnp.float32),
            pltpu.SemaphoreType.DMA,
            pltpu.SemaphoreType.DMA,
            pltpu.SemaphoreType.DMA,
            pltpu.SemaphoreType.DMA,
        ],
    )


def _make_relayout_tc(B, H2, W2, n, g, Bs, first, Hb=112):
    out_shape = jax.ShapeDtypeStruct((B, H2, n, W2), jnp.float32)
    out_spec = pl.BlockSpec((1, Hb, n, W2), lambda b, h: (g * Bs + b, h, 0, 0))
    in_spec = pl.BlockSpec((1, Hb, W2, n), lambda b, h: (b, h, 0, 0))

    if first:
        def tbody0(x_ref, o_ref):
            o_ref[0] = jnp.transpose(x_ref[0], (0, 2, 1))

        return pl.pallas_call(
            tbody0,
            grid=(Bs, H2 // Hb),
            in_specs=[in_spec],
            out_specs=out_spec,
            out_shape=out_shape,
        )

    def tbody(acc_ref, x_ref, o_ref):
        del acc_ref
        o_ref[0] = jnp.transpose(x_ref[0], (0, 2, 1))

    return pl.pallas_call(
        tbody,
        grid=(Bs, H2 // Hb),
        in_specs=[pl.BlockSpec(memory_space=pl.ANY), in_spec],
        out_specs=out_spec,
        out_shape=out_shape,
        input_output_aliases={0: 0},
    )


def kernel(inputs):
    B, H, W, C4 = inputs.shape
    n = C4 // 4
    NW = _NC * _NS
    Bs = next(s for s in range(1, B + 1)
              if B % s == 0 and (s * H) % NW == 0)
    acc = None
    for g in range(B // Bs):
        y = _make_iwt_sc(B, H, W, C4, g * Bs, Bs)(inputs)
        if acc is None:
            acc = _make_relayout_tc(B, 2 * H, 2 * W, n, g, Bs, True)(y)
        else:
            acc = _make_relayout_tc(B, 2 * H, 2 * W, n, g, Bs, False)(acc, y)
    return jnp.transpose(acc, (0, 1, 3, 2))
